# baseline (device time: 117809 ns/iter reference)
import jax
import jax.numpy as jnp
from jax import lax
from jax.experimental import pallas as pl
from jax.experimental.pallas import tpu as pltpu

T_HALF = 512
D = 1024
F = 2048
E_LOC = 4


def _mm(a, b):
    return lax.dot_general(
        a, b, (((1,), (0,)), ((), ())), preferred_element_type=jnp.float32
    )


def _body(x_ref, r_ref, w1_ref, w2_ref, out_ref,
          xall, r_recv, m_all, m_send, p_send, p_recv,
          send_sems, recv_sems):
    my_x = lax.axis_index("x")
    my_y = lax.axis_index("y")
    my_z = lax.axis_index("z")
    nbr = (my_x, 1 - my_y, my_z)
    my0 = my_y * T_HALF

    bsem = pltpu.get_barrier_semaphore()
    pl.semaphore_signal(bsem, inc=1, device_id=nbr,
                        device_id_type=pl.DeviceIdType.MESH)
    pl.semaphore_wait(bsem, 1)

    xall[pl.ds(my0, T_HALF), :] = x_ref[:].astype(jnp.bfloat16)
    rdma_x = pltpu.make_async_remote_copy(
        src_ref=xall.at[pl.ds(my0, T_HALF), :],
        dst_ref=xall.at[pl.ds(my0, T_HALF), :],
        send_sem=send_sems.at[0], recv_sem=recv_sems.at[0],
        device_id=nbr, device_id_type=pl.DeviceIdType.MESH)
    rdma_r = pltpu.make_async_remote_copy(
        src_ref=r_ref, dst_ref=r_recv,
        send_sem=send_sems.at[1], recv_sem=recv_sems.at[1],
        device_id=nbr, device_id_type=pl.DeviceIdType.MESH)
    rdma_x.start()
    rdma_r.start()
    rdma_r.wait()

    g_loc = lax.dot_general(x_ref[:], r_ref[:], (((1,), (0,)), ((), ())),
                            precision=lax.Precision.HIGHEST,
                            preferred_element_type=jnp.float32)
    g_rem = lax.dot_general(x_ref[:], r_recv[:], (((1,), (0,)), ((), ())),
                            precision=lax.Precision.HIGHEST,
                            preferred_element_type=jnp.float32)
    g = jnp.concatenate([g_loc, g_rem], axis=1)
    i8 = lax.broadcasted_iota(jnp.int32, (T_HALF, 8), 1)
    m1 = jnp.max(g, axis=1, keepdims=True)
    p1 = jnp.min(jnp.where(g == m1, i8, 8), axis=1, keepdims=True)
    gm = jnp.where(i8 == p1, -jnp.inf, g)
    m2 = jnp.max(gm, axis=1, keepdims=True)
    p2 = jnp.min(jnp.where(gm == m2, i8, 8), axis=1, keepdims=True)
    e2 = jnp.exp(m2 - m1)
    wa = 1.0 / (1.0 + e2)
    wb = e2 / (1.0 + e2)
    i4 = lax.broadcasted_iota(jnp.int32, (T_HALF, E_LOC), 1)
    m_self = jnp.where(p1 == i4, wa, 0.0) + jnp.where(p2 == i4, wb, 0.0)
    m_out = (jnp.where(p1 == i4 + 4, wa, 0.0)
             + jnp.where(p2 == i4 + 4, wb, 0.0))

    m_all[pl.ds(my0, T_HALF), :] = m_self
    m_send[:] = m_out
    rdma_m = pltpu.make_async_remote_copy(
        src_ref=m_send, dst_ref=m_all.at[pl.ds(my0, T_HALF), :],
        send_sem=send_sems.at[2], recv_sem=recv_sems.at[2],
        device_id=nbr, device_id_type=pl.DeviceIdType.MESH)
    rdma_m.start()
    rdma_x.wait()
    rdma_m.wait()

    xa = xall[:]
    mk = m_all[:]
    p_acc = jnp.zeros((2 * T_HALF, D), jnp.float32)
    for j in range(E_LOC):
        h = jnp.maximum(_mm(xa, w1_ref[j]), 0.0).astype(jnp.bfloat16)
        p_acc = p_acc + _mm(h, w2_ref[j]) * mk[:, j:j + 1]

    p_top = p_acc[:T_HALF]
    p_bot = p_acc[T_HALF:]
    mine = jnp.where(my_y == 0, p_top, p_bot)
    theirs = jnp.where(my_y == 0, p_bot, p_top)

    p_send[:] = theirs.astype(jnp.bfloat16)
    rdma_p = pltpu.make_async_remote_copy(
        src_ref=p_send, dst_ref=p_recv,
        send_sem=send_sems.at[3], recv_sem=recv_sems.at[3],
        device_id=nbr, device_id_type=pl.DeviceIdType.MESH)
    rdma_p.start()
    rdma_p.wait()

    out_ref[:] = mine + p_recv[:].astype(jnp.float32)


def kernel(x, router, W1, W2):
    w1bf = W1.astype(jnp.bfloat16)
    w2bf = W2.astype(jnp.bfloat16)
    return pl.pallas_call(
        _body,
        out_shape=jax.ShapeDtypeStruct((T_HALF, D), jnp.float32),
        in_specs=[pl.BlockSpec(memory_space=pltpu.VMEM)] * 4,
        out_specs=pl.BlockSpec(memory_space=pltpu.VMEM),
        scratch_shapes=[
            pltpu.VMEM((2 * T_HALF, D), jnp.bfloat16),
            pltpu.VMEM((D, E_LOC), jnp.float32),
            pltpu.VMEM((2 * T_HALF, E_LOC), jnp.float32),
            pltpu.VMEM((T_HALF, E_LOC), jnp.float32),
            pltpu.VMEM((T_HALF, D), jnp.bfloat16),
            pltpu.VMEM((T_HALF, D), jnp.bfloat16),
            pltpu.SemaphoreType.DMA((4,)),
            pltpu.SemaphoreType.DMA((4,)),
        ],
        compiler_params=pltpu.CompilerParams(collective_id=0),
    )(x, router, w1bf, w2bf)


# device time: 93657 ns/iter; 1.2579x vs baseline; 1.2579x over previous
import jax
import jax.numpy as jnp
from jax import lax
from jax.experimental import pallas as pl
from jax.experimental.pallas import tpu as pltpu

T_HALF = 512
T_CH = 256
D = 1024
F = 2048
E_LOC = 4

BF = jnp.bfloat16
F32 = jnp.float32


def _mm(a, b):
    return lax.dot_general(
        a, b, (((1,), (0,)), ((), ())), preferred_element_type=F32
    )


def _body(x_ref, r_ref, w1_ref, w2_ref, out_ref,
          r_recv, xc, mc, m_send, c1_send, c1_recv,
          c2_send, c2_recv, c3_send, c3_recv,
          send_sems, recv_sems):
    my_x = lax.axis_index("x")
    my_y = lax.axis_index("y")
    my_z = lax.axis_index("z")
    nbr_y = (my_x, 1 - my_y, my_z)
    nbr_x = (1 - my_x, my_y, my_z)
    nbr_z = (my_x, my_y, 1 - my_z)
    home = my_x == my_y

    bsem = pltpu.get_barrier_semaphore()
    for nbr in (nbr_y, nbr_x, nbr_z):
        pl.semaphore_signal(bsem, inc=1, device_id=nbr,
                            device_id_type=pl.DeviceIdType.MESH)
    pl.semaphore_wait(bsem, 3)

    rdma_r = pltpu.make_async_remote_copy(
        src_ref=r_ref, dst_ref=r_recv,
        send_sem=send_sems.at[0], recv_sem=recv_sems.at[0],
        device_id=nbr_y, device_id_type=pl.DeviceIdType.MESH)

    @pl.when(jnp.logical_not(home))
    def _():
        rdma_r.start()
        rdma_r.wait_send()

    rdma_xc = pltpu.make_async_remote_copy(
        src_ref=xc, dst_ref=xc,
        send_sem=send_sems.at[1], recv_sem=recv_sems.at[1],
        device_id=nbr_y, device_id_type=pl.DeviceIdType.MESH)
    rdma_mc = pltpu.make_async_remote_copy(
        src_ref=m_send, dst_ref=mc,
        send_sem=send_sems.at[2], recv_sem=recv_sems.at[2],
        device_id=nbr_y, device_id_type=pl.DeviceIdType.MESH)

    @pl.when(home)
    def _():
        rdma_r.wait_recv()
        g_loc = lax.dot_general(x_ref[:], r_ref[:], (((1,), (0,)), ((), ())),
                                precision=lax.Precision.HIGHEST,
                                preferred_element_type=F32)
        g_rem = lax.dot_general(x_ref[:], r_recv[:], (((1,), (0,)), ((), ())),
                                precision=lax.Precision.HIGHEST,
                                preferred_element_type=F32)
        g = jnp.concatenate([g_loc, g_rem], axis=1)
        i8 = lax.broadcasted_iota(jnp.int32, (T_HALF, 8), 1)
        m1 = jnp.max(g, axis=1, keepdims=True)
        p1 = jnp.min(jnp.where(g == m1, i8, 8), axis=1, keepdims=True)
        gm = jnp.where(i8 == p1, -jnp.inf, g)
        m2 = jnp.max(gm, axis=1, keepdims=True)
        p2 = jnp.min(jnp.where(gm == m2, i8, 8), axis=1, keepdims=True)
        e2 = jnp.exp(m2 - m1)
        wa = 1.0 / (1.0 + e2)
        wb = e2 / (1.0 + e2)
        i4 = lax.broadcasted_iota(jnp.int32, (T_HALF, E_LOC), 1)
        m_own = jnp.where(p1 == i4, wa, 0.0) + jnp.where(p2 == i4, wb, 0.0)
        m_nbr = (jnp.where(p1 == i4 + 4, wa, 0.0)
                 + jnp.where(p2 == i4 + 4, wb, 0.0))
        xh = x_ref[:].astype(BF)
        xc[:] = jnp.where(my_z == 0, xh[:T_CH], xh[T_CH:])
        mc[:] = jnp.where(my_z == 0, m_own[:T_CH], m_own[T_CH:])
        m_send[:] = jnp.where(my_z == 0, m_nbr[:T_CH], m_nbr[T_CH:])
        rdma_xc.start()
        rdma_mc.start()
        rdma_xc.wait_send()
        rdma_mc.wait_send()

    @pl.when(jnp.logical_not(home))
    def _():
        rdma_xc.wait_recv()
        rdma_mc.wait_recv()

    xa = xc[:]
    mk = mc[:]
    p_acc = jnp.zeros((T_CH, D), F32)
    for j in range(E_LOC):
        h = jnp.maximum(_mm(xa, w1_ref[j]), 0.0).astype(BF)
        p_acc = p_acc + _mm(h, w2_ref[j]) * mk[:, j:j + 1]

    c1_send[:] = p_acc.astype(BF)
    rdma_c1 = pltpu.make_async_remote_copy(
        src_ref=c1_send, dst_ref=c1_recv,
        send_sem=send_sems.at[3], recv_sem=recv_sems.at[3],
        device_id=nbr_y, device_id_type=pl.DeviceIdType.MESH)
    rdma_c1.start()
    rdma_c1.wait()
    s_full = p_acc + c1_recv[:].astype(F32)

    c2_send[:] = s_full.astype(BF)
    rdma_c2 = pltpu.make_async_remote_copy(
        src_ref=c2_send, dst_ref=c2_recv,
        send_sem=send_sems.at[4], recv_sem=recv_sems.at[4],
        device_id=nbr_x, device_id_type=pl.DeviceIdType.MESH)
    rdma_c2.start()
    rdma_c2.wait()
    m1_bf = jnp.where(home, c2_send[:], c2_recv[:])
    m1_f32 = jnp.where(home, s_full, c2_recv[:].astype(F32))

    c3_send[:] = m1_bf
    rdma_c3 = pltpu.make_async_remote_copy(
        src_ref=c3_send, dst_ref=c3_recv,
        send_sem=send_sems.at[5], recv_sem=recv_sems.at[5],
        device_id=nbr_z, device_id_type=pl.DeviceIdType.MESH)
    rdma_c3.start()
    rdma_c3.wait()

    out_ref[pl.ds(T_CH * my_z, T_CH), :] = m1_f32
    out_ref[pl.ds(T_CH * (1 - my_z), T_CH), :] = c3_recv[:].astype(F32)


def kernel(x, router, W1, W2):
    w1bf = W1.astype(BF)
    w2bf = W2.astype(BF)
    return pl.pallas_call(
        _body,
        out_shape=jax.ShapeDtypeStruct((T_HALF, D), F32),
        in_specs=[pl.BlockSpec(memory_space=pltpu.VMEM)] * 4,
        out_specs=pl.BlockSpec(memory_space=pltpu.VMEM),
        scratch_shapes=[
            pltpu.VMEM((D, E_LOC), F32),
            pltpu.VMEM((T_CH, D), BF),
            pltpu.VMEM((T_CH, E_LOC), F32),
            pltpu.VMEM((T_CH, E_LOC), F32),
            pltpu.VMEM((T_CH, D), BF),
            pltpu.VMEM((T_CH, D), BF),
            pltpu.VMEM((T_CH, D), BF),
            pltpu.VMEM((T_CH, D), BF),
            pltpu.VMEM((T_CH, D), BF),
            pltpu.VMEM((T_CH, D), BF),
            pltpu.SemaphoreType.DMA((6,)),
            pltpu.SemaphoreType.DMA((6,)),
        ],
        compiler_params=pltpu.CompilerParams(collective_id=0),
    )(x, router, w1bf, w2bf)


# device time: 62606 ns/iter; 1.8818x vs baseline; 1.4960x over previous
import jax
import jax.numpy as jnp
from jax import lax
from jax.experimental import pallas as pl
from jax.experimental.pallas import tpu as pltpu

T_HALF = 512
T_CH = 256
D = 1024
F = 2048
E_LOC = 4

BF = jnp.bfloat16
F32 = jnp.float32


def _mm(a, b):
    return lax.dot_general(
        a, b, (((1,), (0,)), ((), ())), preferred_element_type=F32
    )


def _body(x_ref, r_ref, w1_ref, w2_ref, out_ref,
          w1v, w2v, r_recv, xc, mc, m_send, c1_send, c1_recv,
          c2_send, c2_recv, c3_send, c3_recv,
          send_sems, recv_sems, wsems):
    my_x = lax.axis_index("x")
    my_y = lax.axis_index("y")
    my_z = lax.axis_index("z")
    nbr_y = (my_x, 1 - my_y, my_z)
    nbr_x = (1 - my_x, my_y, my_z)
    nbr_z = (my_x, my_y, 1 - my_z)
    home = my_x == my_y

    wdma = {}
    for j in range(E_LOC):
        wdma[(0, j)] = pltpu.make_async_copy(
            w1_ref.at[j], w1v.at[j % 2], wsems.at[j % 2])
        wdma[(1, j)] = pltpu.make_async_copy(
            w2_ref.at[j], w2v.at[j % 2], wsems.at[2 + j % 2])
    for j in range(2):
        wdma[(0, j)].start()
        wdma[(1, j)].start()

    bsem = pltpu.get_barrier_semaphore()
    for nbr in (nbr_y, nbr_x, nbr_z):
        pl.semaphore_signal(bsem, inc=1, device_id=nbr,
                            device_id_type=pl.DeviceIdType.MESH)
    pl.semaphore_wait(bsem, 3)

    rdma_r = pltpu.make_async_remote_copy(
        src_ref=r_ref, dst_ref=r_recv,
        send_sem=send_sems.at[0], recv_sem=recv_sems.at[0],
        device_id=nbr_y, device_id_type=pl.DeviceIdType.MESH)
    rdma_xc = pltpu.make_async_remote_copy(
        src_ref=xc, dst_ref=xc,
        send_sem=send_sems.at[1], recv_sem=recv_sems.at[1],
        device_id=nbr_y, device_id_type=pl.DeviceIdType.MESH)
    rdma_mc = pltpu.make_async_remote_copy(
        src_ref=m_send, dst_ref=mc,
        send_sem=send_sems.at[2], recv_sem=recv_sems.at[2],
        device_id=nbr_y, device_id_type=pl.DeviceIdType.MESH)

    @pl.when(jnp.logical_not(home))
    def _():
        rdma_r.start()
        rdma_r.wait_send()

    @pl.when(home)
    def _():
        xc[:] = x_ref[pl.ds(T_CH * my_z, T_CH), :].astype(BF)
        rdma_xc.start()
        g_loc = lax.dot_general(x_ref[:], r_ref[:], (((1,), (0,)), ((), ())),
                                precision=lax.Precision.HIGHEST,
                                preferred_element_type=F32)
        rdma_r.wait_recv()
        g_rem = lax.dot_general(x_ref[:], r_recv[:], (((1,), (0,)), ((), ())),
                                precision=lax.Precision.HIGHEST,
                                preferred_element_type=F32)
        g = jnp.concatenate([g_loc, g_rem], axis=1)
        i8 = lax.broadcasted_iota(jnp.int32, (T_HALF, 8), 1)
        m1 = jnp.max(g, axis=1, keepdims=True)
        p1 = jnp.min(jnp.where(g == m1, i8, 8), axis=1, keepdims=True)
        gm = jnp.where(i8 == p1, -jnp.inf, g)
        m2 = jnp.max(gm, axis=1, keepdims=True)
        p2 = jnp.min(jnp.where(gm == m2, i8, 8), axis=1, keepdims=True)
        e2 = jnp.exp(m2 - m1)
        wa = 1.0 / (1.0 + e2)
        wb = e2 / (1.0 + e2)
        i4 = lax.broadcasted_iota(jnp.int32, (T_HALF, E_LOC), 1)
        m_own = jnp.where(p1 == i4, wa, 0.0) + jnp.where(p2 == i4, wb, 0.0)
        m_nbr = (jnp.where(p1 == i4 + 4, wa, 0.0)
                 + jnp.where(p2 == i4 + 4, wb, 0.0))
        mc[:] = jnp.where(my_z == 0, m_own[:T_CH], m_own[T_CH:])
        m_send[:] = jnp.where(my_z == 0, m_nbr[:T_CH], m_nbr[T_CH:])
        rdma_mc.start()
        rdma_xc.wait_send()
        rdma_mc.wait_send()

    @pl.when(jnp.logical_not(home))
    def _():
        rdma_xc.wait_recv()
        rdma_mc.wait_recv()

    xa = xc[:]
    mk = mc[:]
    p_acc = jnp.zeros((T_CH, D), F32)
    for j in range(E_LOC):
        wdma[(0, j)].wait()
        w1b = w1v[j % 2].astype(BF)
        if j + 2 < E_LOC:
            wdma[(0, j + 2)].start()
        h = jnp.maximum(_mm(xa, w1b), 0.0)
        h = (h * mk[:, j:j + 1]).astype(BF)
        wdma[(1, j)].wait()
        w2b = w2v[j % 2].astype(BF)
        if j + 2 < E_LOC:
            wdma[(1, j + 2)].start()
        p_acc = p_acc + _mm(h, w2b)

    c1_send[:] = p_acc.astype(BF)
    rdma_c1 = pltpu.make_async_remote_copy(
        src_ref=c1_send, dst_ref=c1_recv,
        send_sem=send_sems.at[3], recv_sem=recv_sems.at[3],
        device_id=nbr_y, device_id_type=pl.DeviceIdType.MESH)
    rdma_c1.start()
    rdma_c1.wait()
    s_full = p_acc + c1_recv[:].astype(F32)

    c2_send[:] = s_full.astype(BF)
    rdma_c2 = pltpu.make_async_remote_copy(
        src_ref=c2_send, dst_ref=c2_recv,
        send_sem=send_sems.at[4], recv_sem=recv_sems.at[4],
        device_id=nbr_x, device_id_type=pl.DeviceIdType.MESH)

    @pl.when(home)
    def _():
        rdma_c2.start()
        rdma_c2.wait_send()

    @pl.when(jnp.logical_not(home))
    def _():
        rdma_c2.wait_recv()

    m1_bf = jnp.where(home, c2_send[:], c2_recv[:])
    m1_f32 = jnp.where(home, s_full, c2_recv[:].astype(F32))

    c3_send[:] = m1_bf
    rdma_c3 = pltpu.make_async_remote_copy(
        src_ref=c3_send, dst_ref=c3_recv,
        send_sem=send_sems.at[5], recv_sem=recv_sems.at[5],
        device_id=nbr_z, device_id_type=pl.DeviceIdType.MESH)
    rdma_c3.start()
    rdma_c3.wait()

    out_ref[pl.ds(T_CH * my_z, T_CH), :] = m1_f32
    out_ref[pl.ds(T_CH * (1 - my_z), T_CH), :] = c3_recv[:].astype(F32)


def kernel(x, router, W1, W2):
    return pl.pallas_call(
        _body,
        out_shape=jax.ShapeDtypeStruct((T_HALF, D), F32),
        in_specs=[
            pl.BlockSpec(memory_space=pltpu.VMEM),
            pl.BlockSpec(memory_space=pltpu.VMEM),
            pl.BlockSpec(memory_space=pltpu.MemorySpace.HBM),
            pl.BlockSpec(memory_space=pltpu.MemorySpace.HBM),
        ],
        out_specs=pl.BlockSpec(memory_space=pltpu.VMEM),
        scratch_shapes=[
            pltpu.VMEM((2, D, F), F32),
            pltpu.VMEM((2, F, D), F32),
            pltpu.VMEM((D, E_LOC), F32),
            pltpu.VMEM((T_CH, D), BF),
            pltpu.VMEM((T_CH, E_LOC), F32),
            pltpu.VMEM((T_CH, E_LOC), F32),
            pltpu.VMEM((T_CH, D), BF),
            pltpu.VMEM((T_CH, D), BF),
            pltpu.VMEM((T_CH, D), BF),
            pltpu.VMEM((T_CH, D), BF),
            pltpu.VMEM((T_CH, D), BF),
            pltpu.VMEM((T_CH, D), BF),
            pltpu.SemaphoreType.DMA((6,)),
            pltpu.SemaphoreType.DMA((6,)),
            pltpu.SemaphoreType.DMA((4,)),
        ],
        compiler_params=pltpu.CompilerParams(
            collective_id=0, vmem_limit_bytes=64 * 1024 * 1024),
    )(x, router, W1, W2)


# device time: 62188 ns/iter; 1.8944x vs baseline; 1.0067x over previous
import jax
import jax.numpy as jnp
from jax import lax
from jax.experimental import pallas as pl
from jax.experimental.pallas import tpu as pltpu

T_HALF = 512
T_CH = 256
D = 1024
F = 2048
E_LOC = 4

BF = jnp.bfloat16
F32 = jnp.float32


def _mm(a, b):
    return lax.dot_general(
        a, b, (((1,), (0,)), ((), ())), preferred_element_type=F32
    )


def _body(x_ref, r_ref, w1_ref, w2_ref, out_ref,
          w1v, w2v, r_recv, xc, mc, m_send, c1_send, c1_recv,
          ds_send, dx_recv, dg_recv, dz_recv,
          send_sems, recv_sems, wsems):
    my_x = lax.axis_index("x")
    my_y = lax.axis_index("y")
    my_z = lax.axis_index("z")
    nbr_y = (my_x, 1 - my_y, my_z)
    nbr_x = (1 - my_x, my_y, my_z)
    nbr_z = (my_x, my_y, 1 - my_z)
    diag = (1 - my_x, my_y, 1 - my_z)
    home = my_x == my_y

    wdma = {}
    for j in range(E_LOC):
        s = j % 2
        for hh in range(2):
            wdma[(0, j, hh)] = pltpu.make_async_copy(
                w1_ref.at[j, pl.ds(hh * D // 2, D // 2)],
                w1v.at[s, pl.ds(hh * D // 2, D // 2)],
                wsems.at[4 * s + hh])
            wdma[(1, j, hh)] = pltpu.make_async_copy(
                w2_ref.at[j, pl.ds(hh * F // 2, F // 2)],
                w2v.at[s, pl.ds(hh * F // 2, F // 2)],
                wsems.at[4 * s + 2 + hh])
    for j in range(2):
        for hh in range(2):
            wdma[(0, j, hh)].start()
            wdma[(1, j, hh)].start()

    bsem = pltpu.get_barrier_semaphore()
    for nbr in (nbr_y, nbr_x, nbr_z, diag):
        pl.semaphore_signal(bsem, inc=1, device_id=nbr,
                            device_id_type=pl.DeviceIdType.MESH)
    pl.semaphore_wait(bsem, 4)

    rdma_r = pltpu.make_async_remote_copy(
        src_ref=r_ref, dst_ref=r_recv,
        send_sem=send_sems.at[0], recv_sem=recv_sems.at[0],
        device_id=nbr_y, device_id_type=pl.DeviceIdType.MESH)
    rdma_xc = pltpu.make_async_remote_copy(
        src_ref=xc, dst_ref=xc,
        send_sem=send_sems.at[1], recv_sem=recv_sems.at[1],
        device_id=nbr_y, device_id_type=pl.DeviceIdType.MESH)
    rdma_mc = pltpu.make_async_remote_copy(
        src_ref=m_send, dst_ref=mc,
        send_sem=send_sems.at[2], recv_sem=recv_sems.at[2],
        device_id=nbr_y, device_id_type=pl.DeviceIdType.MESH)

    @pl.when(jnp.logical_not(home))
    def _():
        rdma_r.start()
        rdma_r.wait_send()

    @pl.when(home)
    def _():
        xc[:] = x_ref[pl.ds(T_CH * my_z, T_CH), :].astype(BF)
        rdma_xc.start()
        g_loc = lax.dot_general(x_ref[:], r_ref[:], (((1,), (0,)), ((), ())),
                                precision=lax.Precision.HIGHEST,
                                preferred_element_type=F32)
        rdma_r.wait_recv()
        g_rem = lax.dot_general(x_ref[:], r_recv[:], (((1,), (0,)), ((), ())),
                                precision=lax.Precision.HIGHEST,
                                preferred_element_type=F32)
        g = jnp.concatenate([g_loc, g_rem], axis=1)
        i8 = lax.broadcasted_iota(jnp.int32, (T_HALF, 8), 1)
        m1 = jnp.max(g, axis=1, keepdims=True)
        p1 = jnp.min(jnp.where(g == m1, i8, 8), axis=1, keepdims=True)
        gm = jnp.where(i8 == p1, -jnp.inf, g)
        m2 = jnp.max(gm, axis=1, keepdims=True)
        p2 = jnp.min(jnp.where(gm == m2, i8, 8), axis=1, keepdims=True)
        e2 = jnp.exp(m2 - m1)
        wa = 1.0 / (1.0 + e2)
        wb = e2 / (1.0 + e2)
        i4 = lax.broadcasted_iota(jnp.int32, (T_HALF, E_LOC), 1)
        m_own = jnp.where(p1 == i4, wa, 0.0) + jnp.where(p2 == i4, wb, 0.0)
        m_nbr = (jnp.where(p1 == i4 + 4, wa, 0.0)
                 + jnp.where(p2 == i4 + 4, wb, 0.0))
        mc[:] = jnp.where(my_z == 0, m_own[:T_CH], m_own[T_CH:])
        m_send[:] = jnp.where(my_z == 0, m_nbr[:T_CH], m_nbr[T_CH:])
        rdma_mc.start()

    wdma[(0, 0, 0)].wait()
    w1b0a = w1v[0, :D // 2].astype(BF)
    wdma[(0, 0, 1)].wait()
    w1b0b = w1v[0, D // 2:].astype(BF)
    wdma[(1, 0, 0)].wait()
    w2b0a = w2v[0, :F // 2].astype(BF)
    wdma[(1, 0, 1)].wait()
    w2b0b = w2v[0, F // 2:].astype(BF)

    @pl.when(jnp.logical_not(home))
    def _():
        rdma_xc.wait_recv()
        rdma_mc.wait_recv()

    xa = xc[:]
    mk = mc[:]
    xa0 = xa[:, :D // 2]
    xa1 = xa[:, D // 2:]
    p_acc = jnp.zeros((T_CH, D), F32)
    for j in range(E_LOC):
        s = j % 2
        if j == 0:
            w1ba, w1bb = w1b0a, w1b0b
        else:
            wdma[(0, j, 0)].wait()
            w1ba = w1v[s, :D // 2].astype(BF)
            wdma[(0, j, 1)].wait()
            w1bb = w1v[s, D // 2:].astype(BF)
        if j + 2 < E_LOC:
            for hh in range(2):
                wdma[(0, j + 2, hh)].start()
        h = jnp.maximum(_mm(xa0, w1ba) + _mm(xa1, w1bb), 0.0)
        h = (h * mk[:, j:j + 1]).astype(BF)
        if j == 0:
            w2ba, w2bb = w2b0a, w2b0b
        else:
            wdma[(1, j, 0)].wait()
            w2ba = w2v[s, :F // 2].astype(BF)
            wdma[(1, j, 1)].wait()
            w2bb = w2v[s, F // 2:].astype(BF)
        if j + 2 < E_LOC:
            for hh in range(2):
                wdma[(1, j + 2, hh)].start()
        p_acc = p_acc + _mm(h[:, :F // 2], w2ba) + _mm(h[:, F // 2:], w2bb)

    rdma_c1 = pltpu.make_async_remote_copy(
        src_ref=c1_send, dst_ref=c1_recv,
        send_sem=send_sems.at[3], recv_sem=recv_sems.at[3],
        device_id=nbr_y, device_id_type=pl.DeviceIdType.MESH)

    @pl.when(jnp.logical_not(home))
    def _():
        c1_send[:] = p_acc.astype(BF)
        rdma_c1.start()

    rdma_dx = pltpu.make_async_remote_copy(
        src_ref=ds_send, dst_ref=dx_recv,
        send_sem=send_sems.at[4], recv_sem=recv_sems.at[4],
        device_id=nbr_x, device_id_type=pl.DeviceIdType.MESH)
    rdma_dg = pltpu.make_async_remote_copy(
        src_ref=ds_send, dst_ref=dg_recv,
        send_sem=send_sems.at[5], recv_sem=recv_sems.at[5],
        device_id=diag, device_id_type=pl.DeviceIdType.MESH)
    rdma_dz = pltpu.make_async_remote_copy(
        src_ref=ds_send, dst_ref=dz_recv,
        send_sem=send_sems.at[6], recv_sem=recv_sems.at[6],
        device_id=nbr_z, device_id_type=pl.DeviceIdType.MESH)

    @pl.when(home)
    def _():
        rdma_c1.wait_recv()
        s_full = p_acc + c1_recv[:].astype(F32)
        ds_send[:] = s_full.astype(BF)
        rdma_dx.start()
        rdma_dg.start()
        rdma_dz.start()
        out_ref[pl.ds(T_CH * my_z, T_CH), :] = s_full
        rdma_dz.wait_recv()
        out_ref[pl.ds(T_CH * (1 - my_z), T_CH), :] = dz_recv[:].astype(F32)
        rdma_dx.wait_send()
        rdma_dg.wait_send()
        rdma_dz.wait_send()
        rdma_xc.wait_send()
        rdma_mc.wait_send()

    @pl.when(jnp.logical_not(home))
    def _():
        rdma_c1.wait_send()
        rdma_dx.wait_recv()
        out_ref[pl.ds(T_CH * my_z, T_CH), :] = dx_recv[:].astype(F32)
        rdma_dg.wait_recv()
        out_ref[pl.ds(T_CH * (1 - my_z), T_CH), :] = dg_recv[:].astype(F32)


def kernel(x, router, W1, W2):
    return pl.pallas_call(
        _body,
        out_shape=jax.ShapeDtypeStruct((T_HALF, D), F32),
        in_specs=[
            pl.BlockSpec(memory_space=pltpu.VMEM),
            pl.BlockSpec(memory_space=pltpu.VMEM),
            pl.BlockSpec(memory_space=pltpu.MemorySpace.HBM),
            pl.BlockSpec(memory_space=pltpu.MemorySpace.HBM),
        ],
        out_specs=pl.BlockSpec(memory_space=pltpu.VMEM),
        scratch_shapes=[
            pltpu.VMEM((2, D, F), F32),
            pltpu.VMEM((2, F, D), F32),
            pltpu.VMEM((D, E_LOC), F32),
            pltpu.VMEM((T_CH, D), BF),
            pltpu.VMEM((T_CH, E_LOC), F32),
            pltpu.VMEM((T_CH, E_LOC), F32),
            pltpu.VMEM((T_CH, D), BF),
            pltpu.VMEM((T_CH, D), BF),
            pltpu.VMEM((T_CH, D), BF),
            pltpu.VMEM((T_CH, D), BF),
            pltpu.VMEM((T_CH, D), BF),
            pltpu.VMEM((T_CH, D), BF),
            pltpu.SemaphoreType.DMA((7,)),
            pltpu.SemaphoreType.DMA((7,)),
            pltpu.SemaphoreType.DMA((8,)),
        ],
        compiler_params=pltpu.CompilerParams(
            collective_id=0, vmem_limit_bytes=64 * 1024 * 1024),
    )(x, router, W1, W2)
